# Initial kernel scaffold; baseline (speedup 1.0000x reference)
import jax
import jax.numpy as jnp
from jax.experimental import pallas as pl


def kernel(src, tgt, src_lengths, positions, masks, weights, embed_weights):
    raise NotImplementedError("write your pallas kernel here")


# speedup vs baseline: 2.0101x; 2.0101x over previous
import functools

import jax
import jax.numpy as jnp
from jax import lax
from jax.experimental import pallas as pl
from jax.experimental.pallas import tpu as pltpu
from jax.experimental.pallas import tpu_sc as plsc

V, D, S, K, M = 100000, 128, 2048, 8, 8192
NC, NS, L = 2, 16, 16
NW = NC * NS
ROWS_A = S // NW
WPK = NW // K
KPC = K // NC
RPW = M // WPK
CHUNK = 256
NCHUNK = RPW // CHUNK
GPC = CHUNK // L
DL = D // L
TRASH = S

_mesh = plsc.VectorSubcoreMesh(core_axis_name="c", subcore_axis_name="s")

def _lane_reduce(x, op):
    lanes = lax.iota(jnp.int32, L)
    for sh in (8, 4, 2, 1):
        idx = jnp.bitwise_xor(lanes, sh)
        x = op(x, jnp.take_along_axis(x, idx, axis=0,
                                      mode="promise_in_bounds"))
    return x


@functools.partial(
    pl.kernel,
    out_type=jax.ShapeDtypeStruct((S + 8, D), jnp.float32),
    mesh=_mesh,
    scratch_types=[
        pltpu.VMEM((ROWS_A,), jnp.int32),
        pltpu.VMEM((ROWS_A,), jnp.int32),
        pltpu.VMEM((ROWS_A, D), jnp.float32),
        pltpu.VMEM((M,), jnp.float32),
        pltpu.VMEM((CHUNK,), jnp.int32),
        pltpu.VMEM((CHUNK, D), jnp.float32),
        pltpu.VMEM((NS, D), jnp.float32),
        pltpu.VMEM((K, D), jnp.float32),
        pltpu.VMEM((K,), jnp.int32),
        pltpu.VMEM_SHARED((NS, D), jnp.float32),
        pltpu.SemaphoreType.DMA,
    ],
)
def _sc_combiner(table, src_idx, dst_idx, masks_flat, weights_flat, dstpos16,
                 out, sidx, didx, srows, wbuf, midx, buf, comb, crow,
                 didx8, shared, sem):
    c = lax.axis_index("c")
    s = lax.axis_index("s")
    wid = c * NS + s

    base = wid * ROWS_A
    pltpu.sync_copy(src_idx.at[pl.ds(base, ROWS_A)], sidx)
    pltpu.sync_copy(dst_idx.at[pl.ds(base, ROWS_A)], didx)
    pltpu.async_copy(table.at[sidx], srows, sem).wait()
    pltpu.async_copy(srows, out.at[didx], sem).wait()

    k = c * KPC + s // WPK
    q = s % WPK

    pltpu.sync_copy(weights_flat.at[pl.ds(k * M, M)], wbuf)

    def max_body(i, m):
        return jnp.maximum(m, wbuf[pl.ds(i * L, L)])
    m16 = lax.fori_loop(0, M // L, max_body,
                        jnp.full((L,), -jnp.inf, jnp.float32))
    gmax = _lane_reduce(m16, jnp.maximum)

    def sum_body(i, a):
        return a + jnp.exp(wbuf[pl.ds(i * L, L)] - gmax)
    s16 = lax.fori_loop(0, M // L, sum_body, jnp.zeros((L,), jnp.float32))
    inv = 1.0 / _lane_reduce(s16, jnp.add)

    moff = k * M + q * RPW

    def chunk_body(t, acc8):
        pltpu.sync_copy(masks_flat.at[pl.ds(moff + t * CHUNK, CHUNK)], midx)
        pltpu.async_copy(table.at[midx], buf, sem).wait()

        def group_body(g, acc8):
            w16 = wbuf[pl.ds(q * RPW + t * CHUNK + g * L, L)]
            c16 = jnp.exp(w16 - gmax) * inv
            accs = list(acc8)
            for r in range(L):
                crb = jnp.take_along_axis(
                    c16, jnp.full((L,), r, jnp.int32), axis=0,
                    mode="promise_in_bounds")
                row = g * L + r
                for j in range(DL):
                    accs[j] = accs[j] + crb * buf[row, pl.ds(j * L, L)]
            return tuple(accs)

        return lax.fori_loop(0, GPC, group_body, acc8)

    acc8 = tuple(jnp.zeros((L,), jnp.float32) for _ in range(DL))
    acc8 = lax.fori_loop(0, NCHUNK, chunk_body, acc8)

    for j in range(DL):
        srows[0, pl.ds(j * L, L)] = acc8[j]
    pltpu.sync_copy(srows.at[0], shared.at[s])
    plsc.subcore_barrier()

    @pl.when(s == 0)
    def _():
        pltpu.sync_copy(shared, comb)
        for kl in range(KPC):
            for j in range(DL):
                v = (comb[kl * WPK + 0, pl.ds(j * L, L)]
                     + comb[kl * WPK + 1, pl.ds(j * L, L)]
                     + comb[kl * WPK + 2, pl.ds(j * L, L)]
                     + comb[kl * WPK + 3, pl.ds(j * L, L)])
                crow[kl, pl.ds(j * L, L)] = v
        zero = jnp.zeros((L,), jnp.float32)
        for kl in range(KPC, K):
            for j in range(DL):
                crow[kl, pl.ds(j * L, L)] = zero
        pltpu.sync_copy(dstpos16.at[pl.ds(c * K, K)], didx8)
        pltpu.async_copy(crow, out.at[didx8], sem).wait()


def kernel(src, tgt, src_lengths, positions, masks, weights, embed_weights):
    src_i = src.reshape(S).astype(jnp.int32)
    pos = positions.astype(jnp.int32)
    winner = jnp.concatenate(
        [pos[:-1] != pos[1:], jnp.ones((1,), dtype=bool)])
    dst_pos = jnp.where(winner, pos, TRASH).astype(jnp.int32)
    dst_idx = (jnp.arange(S, dtype=jnp.int32)
               .at[dst_pos].set(TRASH, mode="drop"))
    trash4 = jnp.full((WPK,), TRASH, jnp.int32)
    dstpos16 = jnp.concatenate(
        [dst_pos[:KPC], trash4, dst_pos[KPC:], trash4])
    out = _sc_combiner(
        embed_weights.astype(jnp.float32),
        src_i,
        dst_idx,
        masks.reshape(-1).astype(jnp.int32),
        weights.reshape(-1).astype(jnp.float32),
        dstpos16,
    )
    return (out[:S], tgt, src_lengths)


# speedup vs baseline: 2.4473x; 1.2175x over previous
import functools

import jax
import jax.numpy as jnp
from jax import lax
from jax.experimental import pallas as pl
from jax.experimental.pallas import tpu as pltpu
from jax.experimental.pallas import tpu_sc as plsc

V, D, S, K, M = 100000, 128, 2048, 8, 8192
NC, NS, L = 2, 16, 16
NW = NC * NS
ROWS_A = S // NW
WPK = NW // K
KPC = K // NC
RPW = M // WPK
CHUNK = 256
NCHUNK = RPW // CHUNK
GPC = CHUNK // L
DL = D // L
TRASH = S

_mesh = plsc.VectorSubcoreMesh(core_axis_name="c", subcore_axis_name="s")

def _lane_reduce(x, op):
    lanes = lax.iota(jnp.int32, L)
    for sh in (8, 4, 2, 1):
        idx = jnp.bitwise_xor(lanes, sh)
        x = op(x, jnp.take_along_axis(x, idx, axis=0,
                                      mode="promise_in_bounds"))
    return x


@functools.partial(
    pl.kernel,
    out_type=jax.ShapeDtypeStruct((S + 8, D), jnp.float32),
    mesh=_mesh,
    scratch_types=[
        pltpu.VMEM((ROWS_A,), jnp.int32),
        pltpu.VMEM((ROWS_A,), jnp.int32),
        pltpu.VMEM((ROWS_A, D), jnp.float32),
        pltpu.VMEM((M,), jnp.float32),
        pltpu.VMEM((RPW,), jnp.int32),
        pltpu.VMEM((CHUNK, D), jnp.float32),
        pltpu.VMEM((CHUNK, D), jnp.float32),
        pltpu.VMEM((NS, D), jnp.float32),
        pltpu.VMEM((K, D), jnp.float32),
        pltpu.VMEM((K,), jnp.int32),
        pltpu.VMEM_SHARED((NS, D), jnp.float32),
        pltpu.SemaphoreType.DMA,
        pltpu.SemaphoreType.DMA,
        pltpu.SemaphoreType.DMA,
        pltpu.SemaphoreType.DMA,
        pltpu.SemaphoreType.DMA,
        pltpu.SemaphoreType.DMA,
        pltpu.SemaphoreType.DMA,
        pltpu.SemaphoreType.DMA,
    ],
)
def _sc_combiner(table, src_idx, dst_idx, masks_flat, weights_flat, dstpos16,
                 out, sidx, didx, srows, wbuf, midx_all, buf0, buf1, comb,
                 crow, didx8, shared, sem_w, sem_m, sem_s, sem_d, sem_g,
                 sem_o, sem_b0, sem_b1):
    c = lax.axis_index("c")
    s = lax.axis_index("s")
    wid = c * NS + s
    k = c * KPC + s // WPK
    q = s % WPK
    moff = k * M + q * RPW
    base = wid * ROWS_A

    cp_w = pltpu.async_copy(weights_flat.at[pl.ds(k * M, M)], wbuf, sem_w)
    cp_m = pltpu.async_copy(masks_flat.at[pl.ds(moff, RPW)], midx_all, sem_m)
    cp_si = pltpu.async_copy(src_idx.at[pl.ds(base, ROWS_A)], sidx, sem_s)
    cp_di = pltpu.async_copy(dst_idx.at[pl.ds(base, ROWS_A)], didx, sem_d)

    cp_si.wait()
    cp_sr = pltpu.async_copy(table.at[sidx], srows, sem_g)

    cp_m.wait()
    bufs = [buf0, buf1]
    sems = [sem_b0, sem_b1]
    handles = [
        pltpu.async_copy(table.at[midx_all.at[pl.ds(0, CHUNK)]],
                         buf0, sem_b0),
        pltpu.async_copy(table.at[midx_all.at[pl.ds(CHUNK, CHUNK)]],
                         buf1, sem_b1),
    ]

    cp_w.wait()

    def max_body(i, m):
        return jnp.maximum(m, wbuf[pl.ds(i * L, L)])
    m16 = lax.fori_loop(0, M // L, max_body,
                        jnp.full((L,), -jnp.inf, jnp.float32))
    gmax = _lane_reduce(m16, jnp.maximum)

    def sum_body(i, a):
        return a + jnp.exp(wbuf[pl.ds(i * L, L)] - gmax)
    s16 = lax.fori_loop(0, M // L, sum_body, jnp.zeros((L,), jnp.float32))
    inv = 1.0 / _lane_reduce(s16, jnp.add)

    cp_sr.wait()
    cp_di.wait()
    cp_out = pltpu.async_copy(srows, out.at[didx], sem_o)

    def make_group_body(t, buf):
        def group_body(g, acc8):
            w16 = wbuf[pl.ds(q * RPW + t * CHUNK + g * L, L)]
            c16 = jnp.exp(w16 - gmax) * inv
            accs = list(acc8)
            for r in range(L):
                crb = jnp.take_along_axis(
                    c16, jnp.full((L,), r, jnp.int32), axis=0,
                    mode="promise_in_bounds")
                row = g * L + r
                for j in range(DL):
                    accs[j] = accs[j] + crb * buf[row, pl.ds(j * L, L)]
            return tuple(accs)
        return group_body

    acc8 = tuple(jnp.zeros((L,), jnp.float32) for _ in range(DL))
    for t in range(NCHUNK):
        b = t % 2
        handles[b].wait()
        acc8 = lax.fori_loop(0, GPC, make_group_body(t, bufs[b]), acc8)
        if t + 2 < NCHUNK:
            handles[b] = pltpu.async_copy(
                table.at[midx_all.at[pl.ds((t + 2) * CHUNK, CHUNK)]],
                bufs[b], sems[b])
    cp_out.wait()

    for j in range(DL):
        srows[0, pl.ds(j * L, L)] = acc8[j]
    pltpu.sync_copy(srows.at[0], shared.at[s])
    plsc.subcore_barrier()

    @pl.when(s == 0)
    def _():
        pltpu.sync_copy(shared, comb)
        for kl in range(KPC):
            for j in range(DL):
                v = (comb[kl * WPK + 0, pl.ds(j * L, L)]
                     + comb[kl * WPK + 1, pl.ds(j * L, L)]
                     + comb[kl * WPK + 2, pl.ds(j * L, L)]
                     + comb[kl * WPK + 3, pl.ds(j * L, L)])
                crow[kl, pl.ds(j * L, L)] = v
        zero = jnp.zeros((L,), jnp.float32)
        for kl in range(KPC, K):
            for j in range(DL):
                crow[kl, pl.ds(j * L, L)] = zero
        pltpu.sync_copy(dstpos16.at[pl.ds(c * K, K)], didx8)
        pltpu.async_copy(crow, out.at[didx8], sem_o).wait()


def kernel(src, tgt, src_lengths, positions, masks, weights, embed_weights):
    src_i = src.reshape(S).astype(jnp.int32)
    pos = positions.astype(jnp.int32)
    winner = jnp.concatenate(
        [pos[:-1] != pos[1:], jnp.ones((1,), dtype=bool)])
    dst_pos = jnp.where(winner, pos, TRASH).astype(jnp.int32)
    dst_idx = (jnp.arange(S, dtype=jnp.int32)
               .at[dst_pos].set(TRASH, mode="drop"))
    trash4 = jnp.full((WPK,), TRASH, jnp.int32)
    dstpos16 = jnp.concatenate(
        [dst_pos[:KPC], trash4, dst_pos[KPC:], trash4])
    out = _sc_combiner(
        embed_weights.astype(jnp.float32),
        src_i,
        dst_idx,
        masks.reshape(-1).astype(jnp.int32),
        weights.reshape(-1).astype(jnp.float32),
        dstpos16,
    )
    return (out[:S], tgt, src_lengths)


# speedup vs baseline: 2.5054x; 1.0237x over previous
import functools

import jax
import jax.numpy as jnp
from jax import lax
from jax.experimental import pallas as pl
from jax.experimental.pallas import tpu as pltpu
from jax.experimental.pallas import tpu_sc as plsc

V, D, S, K, M = 100000, 128, 2048, 8, 8192
NC, NS, L = 2, 16, 16
NW = NC * NS
ROWS_A = S // NW
WPK = NW // K
KPC = K // NC
RPW = M // WPK
CHUNK = 256
NCHUNK = RPW // CHUNK
GPC = CHUNK // L
DL = D // L
TRASH = S

_mesh = plsc.VectorSubcoreMesh(core_axis_name="c", subcore_axis_name="s")


def _bcast(x, lane):
    idx = jnp.broadcast_to(jnp.asarray(lane, jnp.int32), (L,))
    return jnp.take_along_axis(x, idx, axis=0, mode="promise_in_bounds")


def _lane_reduce(x, op):
    lanes = lax.iota(jnp.int32, L)
    for sh in (8, 4, 2, 1):
        idx = jnp.bitwise_xor(lanes, sh)
        x = op(x, jnp.take_along_axis(x, idx, axis=0,
                                      mode="promise_in_bounds"))
    return x


@functools.partial(
    pl.kernel,
    out_type=jax.ShapeDtypeStruct((S + 8, D), jnp.float32),
    mesh=_mesh,
    scratch_types=[
        pltpu.VMEM((ROWS_A,), jnp.int32),
        pltpu.VMEM((ROWS_A,), jnp.int32),
        pltpu.VMEM((ROWS_A, D), jnp.float32),
        pltpu.VMEM((M,), jnp.float32),
        pltpu.VMEM((RPW,), jnp.int32),
        pltpu.VMEM((L,), jnp.int32),
        pltpu.VMEM((CHUNK, D), jnp.float32),
        pltpu.VMEM((CHUNK, D), jnp.float32),
        pltpu.VMEM((NS, D), jnp.float32),
        pltpu.VMEM((L, D), jnp.float32),
        pltpu.VMEM((L,), jnp.int32),
        pltpu.VMEM_SHARED((NS, D), jnp.float32),
        pltpu.SemaphoreType.DMA,
        pltpu.SemaphoreType.DMA,
        pltpu.SemaphoreType.DMA,
        pltpu.SemaphoreType.DMA,
        pltpu.SemaphoreType.DMA,
        pltpu.SemaphoreType.DMA,
        pltpu.SemaphoreType.DMA,
        pltpu.SemaphoreType.DMA,
    ],
)
def _sc_combiner(table, src_idx, pos16, masks, weights,
                 out, sidx, didx, srows, wbuf, midx_all, posv, buf0, buf1,
                 comb, crow, didx16, shared, sem_w, sem_m, sem_s, sem_p,
                 sem_g, sem_o, sem_b0, sem_b1):
    c = lax.axis_index("c")
    s = lax.axis_index("s")
    wid = c * NS + s
    k = c * KPC + s // WPK
    q = s % WPK
    base = wid * ROWS_A

    cp_w = pltpu.async_copy(weights.at[k], wbuf, sem_w)
    cp_m = pltpu.async_copy(masks.at[k, pl.ds(q * RPW, RPW)], midx_all, sem_m)
    cp_si = pltpu.async_copy(src_idx.at[pl.ds(base, ROWS_A)], sidx, sem_s)
    cp_p = pltpu.async_copy(pos16, posv, sem_p)

    cp_si.wait()
    cp_sr = pltpu.async_copy(table.at[sidx], srows, sem_g)

    cp_m.wait()
    bufs = [buf0, buf1]
    sems = [sem_b0, sem_b1]
    handles = [
        pltpu.async_copy(table.at[midx_all.at[pl.ds(0, CHUNK)]],
                         buf0, sem_b0),
        pltpu.async_copy(table.at[midx_all.at[pl.ds(CHUNK, CHUNK)]],
                         buf1, sem_b1),
    ]

    cp_p.wait()
    lanes = lax.iota(jnp.int32, L)
    pos = posv[...]
    nxt = jnp.take_along_axis(pos, jnp.minimum(lanes + 1, L - 1), axis=0,
                              mode="promise_in_bounds")
    winner = (lanes < K) & ((pos != nxt) | (lanes == K - 1))
    dstp = jnp.where(winner, pos, TRASH)

    pk_b = [_bcast(dstp, kk) for kk in range(K)]
    for g in range(ROWS_A // L):
        d16 = base + g * L + lanes
        for kk in range(K):
            d16 = jnp.where(d16 == pk_b[kk], TRASH, d16)
        didx[pl.ds(g * L, L)] = d16

    cp_w.wait()

    def max_body(i, m):
        return jnp.maximum(m, wbuf[pl.ds(i * L, L)])
    m16 = plsc.parallel_loop(0, M // L, unroll=8,
                             carry=jnp.full((L,), -jnp.inf, jnp.float32))(
                                 max_body)
    gmax = _lane_reduce(m16, jnp.maximum)

    def sum_body(i, a):
        return a + jnp.exp(wbuf[pl.ds(i * L, L)] - gmax)
    s16 = plsc.parallel_loop(0, M // L, unroll=8,
                             carry=jnp.zeros((L,), jnp.float32))(sum_body)
    inv = 1.0 / _lane_reduce(s16, jnp.add)

    cp_sr.wait()
    cp_out = pltpu.async_copy(srows, out.at[didx], sem_o)

    def make_group_body(t, buf):
        def group_body(g, acc8):
            w16 = wbuf[pl.ds(q * RPW + t * CHUNK + g * L, L)]
            c16 = jnp.exp(w16 - gmax) * inv
            accs = list(acc8)
            for r in range(L):
                crb = _bcast(c16, r)
                row = g * L + r
                for j in range(DL):
                    accs[j] = accs[j] + crb * buf[row, pl.ds(j * L, L)]
            return tuple(accs)
        return group_body

    acc8 = tuple(jnp.zeros((L,), jnp.float32) for _ in range(DL))
    for t in range(NCHUNK):
        b = t % 2
        handles[b].wait()
        acc8 = plsc.parallel_loop(0, GPC, unroll=2, carry=acc8)(
            make_group_body(t, bufs[b]))
        if t + 2 < NCHUNK:
            handles[b] = pltpu.async_copy(
                table.at[midx_all.at[pl.ds((t + 2) * CHUNK, CHUNK)]],
                bufs[b], sems[b])
    cp_out.wait()

    for j in range(DL):
        srows[0, pl.ds(j * L, L)] = acc8[j]
    pltpu.sync_copy(srows.at[0], shared.at[s])
    plsc.subcore_barrier()

    @pl.when(s == 0)
    def _():
        pltpu.sync_copy(shared, comb)
        for kl in range(KPC):
            for j in range(DL):
                v = (comb[kl * WPK + 0, pl.ds(j * L, L)]
                     + comb[kl * WPK + 1, pl.ds(j * L, L)]
                     + comb[kl * WPK + 2, pl.ds(j * L, L)]
                     + comb[kl * WPK + 3, pl.ds(j * L, L)])
                crow[kl, pl.ds(j * L, L)] = v
        zero = jnp.zeros((L,), jnp.float32)
        for kl in range(KPC, L):
            for j in range(DL):
                crow[kl, pl.ds(j * L, L)] = zero
        sel = jnp.where(lanes < KPC, lanes + c * KPC, 0)
        sidx16 = jnp.where(
            lanes < KPC,
            jnp.take_along_axis(dstp, sel, axis=0,
                                mode="promise_in_bounds"),
            TRASH)
        didx16[...] = sidx16
        pltpu.async_copy(crow, out.at[didx16], sem_o).wait()


def kernel(src, tgt, src_lengths, positions, masks, weights, embed_weights):
    pos16 = jnp.concatenate(
        [positions.astype(jnp.int32),
         jnp.full((L - K,), TRASH, jnp.int32)])
    out = _sc_combiner(
        embed_weights.astype(jnp.float32),
        src.reshape(S).astype(jnp.int32),
        pos16,
        masks.astype(jnp.int32),
        weights.astype(jnp.float32),
    )
    return (out[:S], tgt, src_lengths)


# speedup vs baseline: 3.0619x; 1.2222x over previous
import functools

import jax
import jax.numpy as jnp
from jax import lax
from jax.experimental import pallas as pl
from jax.experimental.pallas import tpu as pltpu
from jax.experimental.pallas import tpu_sc as plsc

V, D, S, K, M = 100000, 128, 2048, 8, 8192
NC, NS, L = 2, 16, 16
NW = NC * NS
ROWS_A = S // NW
WPK = NW // K
KPC = K // NC
RPW = M // WPK
CHUNK = 256
NCHUNK = RPW // CHUNK
GPC = CHUNK // L
DL = D // L
TRASH = S

_mesh = plsc.VectorSubcoreMesh(core_axis_name="c", subcore_axis_name="s")


def _bcast(x, lane):
    idx = jnp.broadcast_to(jnp.asarray(lane, jnp.int32), (L,))
    return jnp.take_along_axis(x, idx, axis=0, mode="promise_in_bounds")


def _lane_reduce(x, op):
    lanes = lax.iota(jnp.int32, L)
    for sh in (8, 4, 2, 1):
        idx = jnp.bitwise_xor(lanes, sh)
        x = op(x, jnp.take_along_axis(x, idx, axis=0,
                                      mode="promise_in_bounds"))
    return x


@functools.partial(
    pl.kernel,
    out_type=jax.ShapeDtypeStruct((S + 8, D), jnp.float32),
    mesh=_mesh,
    scratch_types=[
        pltpu.VMEM((ROWS_A,), jnp.int32),
        pltpu.VMEM((ROWS_A,), jnp.int32),
        pltpu.VMEM((ROWS_A, D), jnp.float32),
        pltpu.VMEM((M,), jnp.float32),
        pltpu.VMEM((RPW,), jnp.int32),
        pltpu.VMEM((RPW,), jnp.float32),
        pltpu.VMEM((L,), jnp.int32),
        pltpu.VMEM((CHUNK, D), jnp.float32),
        pltpu.VMEM((CHUNK, D), jnp.float32),
        pltpu.VMEM((NS, D), jnp.float32),
        pltpu.VMEM((L, D), jnp.float32),
        pltpu.VMEM((L,), jnp.int32),
        pltpu.VMEM_SHARED((NS, D), jnp.float32),
        pltpu.SemaphoreType.DMA,
        pltpu.SemaphoreType.DMA,
        pltpu.SemaphoreType.DMA,
        pltpu.SemaphoreType.DMA,
        pltpu.SemaphoreType.DMA,
        pltpu.SemaphoreType.DMA,
        pltpu.SemaphoreType.DMA,
        pltpu.SemaphoreType.DMA,
    ],
)
def _sc_combiner(table, src_idx, pos16, masks, weights,
                 out, sidx, didx, srows, wbuf, midx_all, cbuf, posv,
                 buf0, buf1, comb, crow, didx16, shared, sem_w, sem_m,
                 sem_s, sem_p, sem_g, sem_o, sem_b0, sem_b1):
    c = lax.axis_index("c")
    s = lax.axis_index("s")
    wid = c * NS + s
    k = c * KPC + s // WPK
    q = s % WPK
    base = wid * ROWS_A

    cp_w = pltpu.async_copy(weights.at[k], wbuf, sem_w)
    cp_m = pltpu.async_copy(masks.at[k, pl.ds(q * RPW, RPW)], midx_all, sem_m)
    cp_si = pltpu.async_copy(src_idx.at[pl.ds(base, ROWS_A)], sidx, sem_s)
    cp_p = pltpu.async_copy(pos16, posv, sem_p)

    cp_si.wait()
    cp_sr = pltpu.async_copy(table.at[sidx], srows, sem_g)

    cp_m.wait()
    bufs = [buf0, buf1]
    sems = [sem_b0, sem_b1]
    handles = [
        pltpu.async_copy(table.at[midx_all.at[pl.ds(0, CHUNK)]],
                         buf0, sem_b0),
        pltpu.async_copy(table.at[midx_all.at[pl.ds(CHUNK, CHUNK)]],
                         buf1, sem_b1),
    ]

    cp_p.wait()
    lanes = lax.iota(jnp.int32, L)
    pos = posv[...]
    nxt = jnp.take_along_axis(pos, jnp.minimum(lanes + 1, L - 1), axis=0,
                              mode="promise_in_bounds")
    winner = (lanes < K) & ((pos != nxt) | (lanes == K - 1))
    dstp = jnp.where(winner, pos, TRASH)

    pk_b = [_bcast(dstp, kk) for kk in range(K)]
    for g in range(ROWS_A // L):
        d16 = base + g * L + lanes
        for kk in range(K):
            d16 = jnp.where(d16 == pk_b[kk], TRASH, d16)
        didx[pl.ds(g * L, L)] = d16

    cp_w.wait()

    def max_body(i, m):
        return jnp.maximum(m, wbuf[pl.ds(i * L, L)])
    m16 = plsc.parallel_loop(0, M // L, unroll=8,
                             carry=jnp.full((L,), -jnp.inf, jnp.float32))(
                                 max_body)
    gmax = _lane_reduce(m16, jnp.maximum)

    def sum_body(i, a):
        return a + jnp.exp(wbuf[pl.ds(i * L, L)] - gmax)
    s16 = plsc.parallel_loop(0, M // L, unroll=8,
                             carry=jnp.zeros((L,), jnp.float32))(sum_body)
    inv = 1.0 / _lane_reduce(s16, jnp.add)

    @plsc.parallel_loop(0, RPW // L, unroll=4)
    def _coef(i):
        w16 = wbuf[pl.ds(q * RPW + i * L, L)]
        cbuf[pl.ds(i * L, L)] = jnp.exp(w16 - gmax) * inv

    cp_sr.wait()
    cp_out = pltpu.async_copy(srows, out.at[didx], sem_o)

    def make_row_body(t, buf):
        def row_body(r, acc8):
            g = t * CHUNK + (r & ~(L - 1))
            c16 = cbuf[pl.ds(g, L)]
            crb = _bcast(c16, r & (L - 1))
            accs = list(acc8)
            for j in range(DL):
                accs[j] = accs[j] + crb * buf[r, pl.ds(j * L, L)]
            return tuple(accs)
        return row_body

    acc8 = tuple(jnp.zeros((L,), jnp.float32) for _ in range(DL))
    for t in range(NCHUNK):
        b = t % 2
        handles[b].wait()
        acc8 = plsc.parallel_loop(0, CHUNK, unroll=4, carry=acc8)(
            make_row_body(t, bufs[b]))
        if t + 2 < NCHUNK:
            handles[b] = pltpu.async_copy(
                table.at[midx_all.at[pl.ds((t + 2) * CHUNK, CHUNK)]],
                bufs[b], sems[b])
    cp_out.wait()

    for j in range(DL):
        srows[0, pl.ds(j * L, L)] = acc8[j]
    pltpu.sync_copy(srows.at[0], shared.at[s])
    plsc.subcore_barrier()

    @pl.when(s == 0)
    def _():
        pltpu.sync_copy(shared, comb)
        for kl in range(KPC):
            for j in range(DL):
                v = (comb[kl * WPK + 0, pl.ds(j * L, L)]
                     + comb[kl * WPK + 1, pl.ds(j * L, L)]
                     + comb[kl * WPK + 2, pl.ds(j * L, L)]
                     + comb[kl * WPK + 3, pl.ds(j * L, L)])
                crow[kl, pl.ds(j * L, L)] = v
        zero = jnp.zeros((L,), jnp.float32)
        for kl in range(KPC, L):
            for j in range(DL):
                crow[kl, pl.ds(j * L, L)] = zero
        sel = jnp.where(lanes < KPC, lanes + c * KPC, 0)
        sidx16 = jnp.where(
            lanes < KPC,
            jnp.take_along_axis(dstp, sel, axis=0,
                                mode="promise_in_bounds"),
            TRASH)
        didx16[...] = sidx16
        pltpu.async_copy(crow, out.at[didx16], sem_o).wait()


def kernel(src, tgt, src_lengths, positions, masks, weights, embed_weights):
    pos16 = jnp.concatenate(
        [positions.astype(jnp.int32),
         jnp.full((L - K,), TRASH, jnp.int32)])
    out = _sc_combiner(
        embed_weights.astype(jnp.float32),
        src.reshape(S).astype(jnp.int32),
        pos16,
        masks.astype(jnp.int32),
        weights.astype(jnp.float32),
    )
    return (out[:S], tgt, src_lengths)


# speedup vs baseline: 3.1490x; 1.0284x over previous
import functools

import jax
import jax.numpy as jnp
from jax import lax
from jax.experimental import pallas as pl
from jax.experimental.pallas import tpu as pltpu
from jax.experimental.pallas import tpu_sc as plsc

V, D, S, K, M = 100000, 128, 2048, 8, 8192
NC, NS, L = 2, 16, 16
NW = NC * NS
ROWS_A = S // NW
WPK = NW // K
KPC = K // NC
RPW = M // WPK
CHUNK = 256
NCHUNK = RPW // CHUNK
GPC = CHUNK // L
DL = D // L
TRASH = S

_mesh = plsc.VectorSubcoreMesh(core_axis_name="c", subcore_axis_name="s")


def _bcast(x, lane):
    idx = jnp.broadcast_to(jnp.asarray(lane, jnp.int32), (L,))
    return jnp.take_along_axis(x, idx, axis=0, mode="promise_in_bounds")


def _lane_reduce(x, op):
    lanes = lax.iota(jnp.int32, L)
    for sh in (8, 4, 2, 1):
        idx = jnp.bitwise_xor(lanes, sh)
        x = op(x, jnp.take_along_axis(x, idx, axis=0,
                                      mode="promise_in_bounds"))
    return x


@functools.partial(
    pl.kernel,
    out_type=jax.ShapeDtypeStruct((S + 8, D), jnp.float32),
    mesh=_mesh,
    scratch_types=[
        pltpu.VMEM((ROWS_A,), jnp.int32),
        pltpu.VMEM((ROWS_A,), jnp.int32),
        pltpu.VMEM((ROWS_A, D), jnp.float32),
        pltpu.VMEM((M,), jnp.float32),
        pltpu.VMEM((RPW,), jnp.int32),
        pltpu.VMEM((RPW,), jnp.float32),
        pltpu.VMEM((L,), jnp.int32),
        pltpu.VMEM((CHUNK, D), jnp.float32),
        pltpu.VMEM((CHUNK, D), jnp.float32),
        pltpu.VMEM((NS, D), jnp.float32),
        pltpu.VMEM((L, D), jnp.float32),
        pltpu.VMEM((L,), jnp.int32),
        pltpu.VMEM_SHARED((NS, D), jnp.float32),
        pltpu.SemaphoreType.DMA,
        pltpu.SemaphoreType.DMA,
        pltpu.SemaphoreType.DMA,
        pltpu.SemaphoreType.DMA,
        pltpu.SemaphoreType.DMA,
        pltpu.SemaphoreType.DMA,
        pltpu.SemaphoreType.DMA,
        pltpu.SemaphoreType.DMA,
    ],
)
def _sc_combiner(table, src_idx, pos16, masks, weights,
                 out, sidx, didx, srows, wbuf, midx_all, cbuf, posv,
                 buf0, buf1, comb, crow, didx16, shared, sem_w, sem_m,
                 sem_s, sem_p, sem_g, sem_o, sem_b0, sem_b1):
    c = lax.axis_index("c")
    s = lax.axis_index("s")
    wid = c * NS + s
    k = c * KPC + s // WPK
    q = s % WPK
    base = wid * ROWS_A

    cp_w = pltpu.async_copy(weights.at[k], wbuf, sem_w)
    cp_m = pltpu.async_copy(masks.at[k, pl.ds(q * RPW, RPW)], midx_all, sem_m)
    cp_si = pltpu.async_copy(src_idx.at[pl.ds(base, ROWS_A)], sidx, sem_s)
    cp_p = pltpu.async_copy(pos16, posv.at[pl.ds(0, K)], sem_p)

    cp_si.wait()
    cp_sr = pltpu.async_copy(table.at[sidx], srows, sem_g)

    cp_m.wait()
    bufs = [buf0, buf1]
    sems = [sem_b0, sem_b1]
    handles = [
        pltpu.async_copy(table.at[midx_all.at[pl.ds(0, CHUNK)]],
                         buf0, sem_b0),
        pltpu.async_copy(table.at[midx_all.at[pl.ds(CHUNK, CHUNK)]],
                         buf1, sem_b1),
    ]

    cp_p.wait()
    lanes = lax.iota(jnp.int32, L)
    pos = jnp.where(lanes < K, posv[...], TRASH)
    nxt = jnp.take_along_axis(pos, jnp.minimum(lanes + 1, L - 1), axis=0,
                              mode="promise_in_bounds")
    winner = (lanes < K) & ((pos != nxt) | (lanes == K - 1))
    dstp = jnp.where(winner, pos, TRASH)

    pk_b = [_bcast(dstp, kk) for kk in range(K)]
    for g in range(ROWS_A // L):
        d16 = base + g * L + lanes
        for kk in range(K):
            d16 = jnp.where(d16 == pk_b[kk], TRASH, d16)
        didx[pl.ds(g * L, L)] = d16

    cp_w.wait()

    def max_body(i, m):
        return jnp.maximum(m, wbuf[pl.ds(i * L, L)])
    m16 = plsc.parallel_loop(0, M // L, unroll=8,
                             carry=jnp.full((L,), -jnp.inf, jnp.float32))(
                                 max_body)
    gmax = _lane_reduce(m16, jnp.maximum)

    def sum_body(i, a):
        return a + jnp.exp(wbuf[pl.ds(i * L, L)] - gmax)
    s16 = plsc.parallel_loop(0, M // L, unroll=8,
                             carry=jnp.zeros((L,), jnp.float32))(sum_body)
    inv = 1.0 / _lane_reduce(s16, jnp.add)

    @plsc.parallel_loop(0, RPW // L, unroll=4)
    def _coef(i):
        w16 = wbuf[pl.ds(q * RPW + i * L, L)]
        cbuf[pl.ds(i * L, L)] = jnp.exp(w16 - gmax) * inv

    cp_sr.wait()
    cp_out = pltpu.async_copy(srows, out.at[didx], sem_o)

    def make_row_body(t, buf):
        def row_body(r, acc8):
            g = t * CHUNK + (r & ~(L - 1))
            c16 = cbuf[pl.ds(g, L)]
            crb = _bcast(c16, r & (L - 1))
            accs = list(acc8)
            for j in range(DL):
                accs[j] = accs[j] + crb * buf[r, pl.ds(j * L, L)]
            return tuple(accs)
        return row_body

    acc8 = tuple(jnp.zeros((L,), jnp.float32) for _ in range(DL))
    for t in range(NCHUNK):
        b = t % 2
        handles[b].wait()
        acc8 = plsc.parallel_loop(0, CHUNK, unroll=4, carry=acc8)(
            make_row_body(t, bufs[b]))
        if t + 2 < NCHUNK:
            handles[b] = pltpu.async_copy(
                table.at[midx_all.at[pl.ds((t + 2) * CHUNK, CHUNK)]],
                bufs[b], sems[b])
    cp_out.wait()

    for j in range(DL):
        srows[0, pl.ds(j * L, L)] = acc8[j]
    pltpu.sync_copy(srows.at[0], shared.at[s])
    plsc.subcore_barrier()

    @pl.when(s == 0)
    def _():
        pltpu.sync_copy(shared, comb)
        for kl in range(KPC):
            for j in range(DL):
                v = (comb[kl * WPK + 0, pl.ds(j * L, L)]
                     + comb[kl * WPK + 1, pl.ds(j * L, L)]
                     + comb[kl * WPK + 2, pl.ds(j * L, L)]
                     + comb[kl * WPK + 3, pl.ds(j * L, L)])
                crow[kl, pl.ds(j * L, L)] = v
        zero = jnp.zeros((L,), jnp.float32)
        for kl in range(KPC, L):
            for j in range(DL):
                crow[kl, pl.ds(j * L, L)] = zero
        sel = jnp.where(lanes < KPC, lanes + c * KPC, 0)
        sidx16 = jnp.where(
            lanes < KPC,
            jnp.take_along_axis(dstp, sel, axis=0,
                                mode="promise_in_bounds"),
            TRASH)
        didx16[...] = sidx16
        pltpu.async_copy(crow, out.at[didx16], sem_o).wait()


def kernel(src, tgt, src_lengths, positions, masks, weights, embed_weights):
    out = _sc_combiner(
        embed_weights.astype(jnp.float32),
        src.reshape(S).astype(jnp.int32),
        positions.astype(jnp.int32),
        masks.astype(jnp.int32),
        weights.astype(jnp.float32),
    )
    return (out[:S], tgt, src_lengths)


# speedup vs baseline: 3.2916x; 1.0453x over previous
import functools

import jax
import jax.numpy as jnp
from jax import lax
from jax.experimental import pallas as pl
from jax.experimental.pallas import tpu as pltpu
from jax.experimental.pallas import tpu_sc as plsc

V, D, S, K, M = 100000, 128, 2048, 8, 8192
NC, NS, L = 2, 16, 16
NW = NC * NS
ROWS_A = S // NW
WPK = NW // K
KPC = K // NC
RPW = M // WPK
CHUNK = 128
NBUF = 4
NCHUNK = RPW // CHUNK
GPC = CHUNK // L
DL = D // L
TRASH = S

_mesh = plsc.VectorSubcoreMesh(core_axis_name="c", subcore_axis_name="s")


def _bcast(x, lane):
    idx = jnp.broadcast_to(jnp.asarray(lane, jnp.int32), (L,))
    return jnp.take_along_axis(x, idx, axis=0, mode="promise_in_bounds")


def _lane_reduce(x, op):
    lanes = lax.iota(jnp.int32, L)
    for sh in (8, 4, 2, 1):
        idx = jnp.bitwise_xor(lanes, sh)
        x = op(x, jnp.take_along_axis(x, idx, axis=0,
                                      mode="promise_in_bounds"))
    return x


@functools.partial(
    pl.kernel,
    out_type=jax.ShapeDtypeStruct((S + 8, D), jnp.float32),
    mesh=_mesh,
    scratch_types=[
        pltpu.VMEM((ROWS_A,), jnp.int32),
        pltpu.VMEM((ROWS_A,), jnp.int32),
        pltpu.VMEM((ROWS_A, D), jnp.float32),
        pltpu.VMEM((M,), jnp.float32),
        pltpu.VMEM((RPW,), jnp.int32),
        pltpu.VMEM((RPW,), jnp.float32),
        pltpu.VMEM((L,), jnp.int32),
        pltpu.VMEM((CHUNK, D), jnp.float32),
        pltpu.VMEM((CHUNK, D), jnp.float32),
        pltpu.VMEM((CHUNK, D), jnp.float32),
        pltpu.VMEM((CHUNK, D), jnp.float32),
        pltpu.VMEM((NS, D), jnp.float32),
        pltpu.VMEM((L, D), jnp.float32),
        pltpu.VMEM((L,), jnp.int32),
        pltpu.VMEM_SHARED((NS, D), jnp.float32),
        pltpu.SemaphoreType.DMA,
        pltpu.SemaphoreType.DMA,
        pltpu.SemaphoreType.DMA,
        pltpu.SemaphoreType.DMA,
        pltpu.SemaphoreType.DMA,
        pltpu.SemaphoreType.DMA,
        pltpu.SemaphoreType.DMA,
        pltpu.SemaphoreType.DMA,
        pltpu.SemaphoreType.DMA,
        pltpu.SemaphoreType.DMA,
    ],
)
def _sc_combiner(table, src_idx, pos16, masks, weights,
                 out, sidx, didx, srows, wbuf, midx_all, cbuf, posv,
                 buf0, buf1, buf2, buf3, comb, crow, didx16, shared,
                 sem_w, sem_m, sem_s, sem_p, sem_g, sem_o,
                 sem_b0, sem_b1, sem_b2, sem_b3):
    c = lax.axis_index("c")
    s = lax.axis_index("s")
    wid = c * NS + s
    k = c * KPC + s // WPK
    q = s % WPK
    base = wid * ROWS_A

    cp_w = pltpu.async_copy(weights.at[k], wbuf, sem_w)
    cp_m = pltpu.async_copy(masks.at[k, pl.ds(q * RPW, RPW)], midx_all, sem_m)
    cp_si = pltpu.async_copy(src_idx.at[pl.ds(base, ROWS_A)], sidx, sem_s)
    cp_p = pltpu.async_copy(pos16, posv.at[pl.ds(0, K)], sem_p)

    cp_si.wait()
    cp_sr = pltpu.async_copy(table.at[sidx], srows, sem_g)

    cp_m.wait()
    bufs = [buf0, buf1, buf2, buf3]
    sems = [sem_b0, sem_b1, sem_b2, sem_b3]
    handles = [
        pltpu.async_copy(table.at[midx_all.at[pl.ds(t * CHUNK, CHUNK)]],
                         bufs[t], sems[t])
        for t in range(NBUF)
    ]

    cp_p.wait()
    lanes = lax.iota(jnp.int32, L)
    pos = jnp.where(lanes < K, posv[...], TRASH)
    nxt = jnp.take_along_axis(pos, jnp.minimum(lanes + 1, L - 1), axis=0,
                              mode="promise_in_bounds")
    winner = (lanes < K) & ((pos != nxt) | (lanes == K - 1))
    dstp = jnp.where(winner, pos, TRASH)

    pk_b = [_bcast(dstp, kk) for kk in range(K)]
    for g in range(ROWS_A // L):
        d16 = base + g * L + lanes
        for kk in range(K):
            d16 = jnp.where(d16 == pk_b[kk], TRASH, d16)
        didx[pl.ds(g * L, L)] = d16

    cp_w.wait()

    def max_body(i, m):
        return jnp.maximum(m, wbuf[pl.ds(i * L, L)])
    m16 = plsc.parallel_loop(0, M // L, unroll=8,
                             carry=jnp.full((L,), -jnp.inf, jnp.float32))(
                                 max_body)
    gmax = _lane_reduce(m16, jnp.maximum)

    def sum_body(i, a):
        return a + jnp.exp(wbuf[pl.ds(i * L, L)] - gmax)
    s16 = plsc.parallel_loop(0, M // L, unroll=8,
                             carry=jnp.zeros((L,), jnp.float32))(sum_body)
    inv = 1.0 / _lane_reduce(s16, jnp.add)

    @plsc.parallel_loop(0, RPW // L, unroll=4)
    def _coef(i):
        w16 = wbuf[pl.ds(q * RPW + i * L, L)]
        cbuf[pl.ds(i * L, L)] = jnp.exp(w16 - gmax) * inv

    cp_sr.wait()
    cp_out = pltpu.async_copy(srows, out.at[didx], sem_o)

    def make_row_body(t, buf):
        def row_body(r, acc8):
            g = t * CHUNK + (r & ~(L - 1))
            c16 = cbuf[pl.ds(g, L)]
            crb = _bcast(c16, r & (L - 1))
            accs = list(acc8)
            for j in range(DL):
                accs[j] = accs[j] + crb * buf[r, pl.ds(j * L, L)]
            return tuple(accs)
        return row_body

    acc8 = tuple(jnp.zeros((L,), jnp.float32) for _ in range(DL))
    for t in range(NCHUNK):
        b = t % NBUF
        handles[b].wait()
        acc8 = plsc.parallel_loop(0, CHUNK, unroll=4, carry=acc8)(
            make_row_body(t, bufs[b]))
        if t + NBUF < NCHUNK:
            handles[b] = pltpu.async_copy(
                table.at[midx_all.at[pl.ds((t + NBUF) * CHUNK, CHUNK)]],
                bufs[b], sems[b])
    cp_out.wait()

    for j in range(DL):
        srows[0, pl.ds(j * L, L)] = acc8[j]
    pltpu.sync_copy(srows.at[0], shared.at[s])
    plsc.subcore_barrier()

    @pl.when(s == 0)
    def _():
        pltpu.sync_copy(shared, comb)
        for kl in range(KPC):
            for j in range(DL):
                v = (comb[kl * WPK + 0, pl.ds(j * L, L)]
                     + comb[kl * WPK + 1, pl.ds(j * L, L)]
                     + comb[kl * WPK + 2, pl.ds(j * L, L)]
                     + comb[kl * WPK + 3, pl.ds(j * L, L)])
                crow[kl, pl.ds(j * L, L)] = v
        zero = jnp.zeros((L,), jnp.float32)
        for kl in range(KPC, L):
            for j in range(DL):
                crow[kl, pl.ds(j * L, L)] = zero
        sel = jnp.where(lanes < KPC, lanes + c * KPC, 0)
        sidx16 = jnp.where(
            lanes < KPC,
            jnp.take_along_axis(dstp, sel, axis=0,
                                mode="promise_in_bounds"),
            TRASH)
        didx16[...] = sidx16
        pltpu.async_copy(crow, out.at[didx16], sem_o).wait()


def kernel(src, tgt, src_lengths, positions, masks, weights, embed_weights):
    out = _sc_combiner(
        embed_weights.astype(jnp.float32),
        src.reshape(S).astype(jnp.int32),
        positions.astype(jnp.int32),
        masks.astype(jnp.int32),
        weights.astype(jnp.float32),
    )
    return (out[:S], tgt, src_lengths)
